# baseline (speedup 1.0000x reference)
import functools

import jax
import jax.numpy as jnp
from jax import lax
from jax.experimental import pallas as pl
from jax.experimental.pallas import tpu as pltpu
from jax.experimental.pallas import tpu_sc as plsc

N = 10000
D = 128
NNZ = 320000
NC = 2
NS = 16
NW = NC * NS
K = 128
G = 8
CH = -(-NNZ // (NW * K * G)) * G
PER_W = CH * K
PAD = PER_W * NW
RPS = 632
NP = RPS * NS
HR = NP // 128
L = 16


def _sc_phase(table, gidx, sidx, z128):
    mesh = plsc.VectorSubcoreMesh(core_axis_name="c", subcore_axis_name="s")

    @functools.partial(
        pl.kernel,
        out_type=[
            jax.ShapeDtypeStruct((NC, NP, D), jnp.float32),
            jax.ShapeDtypeStruct((NW, HR, 128), jnp.float32),
        ],
        mesh=mesh,
        scratch_types=[
            pltpu.VMEM((G, K), jnp.int32),
            pltpu.VMEM((G, K), jnp.int32),
            pltpu.VMEM((K, D), jnp.float32),
            pltpu.VMEM((HR, 128), jnp.float32),
            pltpu.VMEM_SHARED((NP + 8, D), jnp.float32),
            pltpu.SemaphoreType.DMA,
        ],
        compiler_params=pltpu.CompilerParams(needs_layout_passes=False),
    )
    def phase(tab_hbm, gidx_hbm, sidx_hbm, z128_hbm,
              acc_out, hist_out, gi_v, si_v, rows_v, hist_v, acc_sp, sem):
        c = lax.axis_index("c")
        s = lax.axis_index("s")
        w = s * NC + c
        base = s * RPS
        pltpu.sync_copy(z128_hbm, acc_sp.at[pl.ds(base, RPS)])

        zv = jnp.zeros((L,), jnp.float32)

        def zrow(r, carry):
            for l in range(128 // L):
                hist_v[r, pl.ds(l * L, L)] = zv
            return carry

        lax.fori_loop(0, HR, zrow, 0)
        plsc.subcore_barrier()

        def group(g, carry):
            pltpu.sync_copy(gidx_hbm.at[w, pl.ds(g * G, G)], gi_v)
            pltpu.sync_copy(sidx_hbm.at[w, pl.ds(g * G, G)], si_v)
            for j in range(G):
                pltpu.async_copy(tab_hbm.at[gi_v.at[j]], rows_v, sem).wait()
                pltpu.sync_copy(rows_v, acc_sp.at[si_v.at[j]], add=True)
                for u in range(K // L):
                    v = si_v[j, pl.ds(u * L, L)]
                    cnt, last = plsc.scan_count(v)
                    r = lax.shift_right_logical(v, 7)
                    col = lax.bitwise_and(v, 127)
                    plsc.addupdate_scatter(
                        hist_v, [r, col], cnt.astype(jnp.float32), mask=last)
            return carry

        lax.fori_loop(0, CH // G, group, 0)
        plsc.subcore_barrier()
        pltpu.sync_copy(acc_sp.at[pl.ds(base, RPS)],
                        acc_out.at[c, pl.ds(base, RPS)])
        pltpu.sync_copy(hist_v, hist_out.at[w])

    return phase(table, gidx, sidx, z128)


def _combine(acc_p, hist_p, bias2d, final):
    def body(a_ref, h_ref, b_ref, o_ref):
        sums = a_ref[0] + a_ref[1]
        cnt = jnp.sum(h_ref[0], axis=0)
        rec = jnp.where(cnt == 0.0, 0.0,
                        1.0 / jnp.where(cnt == 0.0, 1.0, cnt))
        ri = lax.broadcasted_iota(jnp.int32, (128, 128), 0)
        ci = lax.broadcasted_iota(jnp.int32, (128, 128), 1)
        dg = jnp.where(ri == ci, rec[None, :], 0.0)
        y = jax.lax.dot(dg, sums,
                        precision=jax.lax.Precision.HIGHEST,
                        preferred_element_type=jnp.float32)
        if final:
            y = y + b_ref[...]
            y = jnp.where(y >= 0.0, y, 0.01 * y)
        o_ref[...] = y

    return pl.pallas_call(
        body,
        grid=(HR,),
        in_specs=[
            pl.BlockSpec((NC, 128, D), lambda i: (0, i, 0)),
            pl.BlockSpec((1, NW, 128), lambda i: (i, 0, 0)),
            pl.BlockSpec((1, D), lambda i: (0, 0)),
        ],
        out_specs=pl.BlockSpec((128, D), lambda i: (i, 0)),
        out_shape=jax.ShapeDtypeStruct((NP, D), jnp.float32),
    )(acc_p, hist_p, bias2d)


def kernel(x, hyperedge_index, bias):
    src = hyperedge_index[0].astype(jnp.int32)
    dst = hyperedge_index[1].astype(jnp.int32)
    npad = PAD - NNZ
    pad_s = N + (jnp.arange(npad, dtype=jnp.int32) % (NP - N))
    pad_g = jnp.zeros((npad,), jnp.int32)
    src_g = jnp.concatenate([src, pad_g]).reshape(NW, CH, K)
    src_s = jnp.concatenate([src, pad_s]).reshape(NW, CH, K)
    dst_g = jnp.concatenate([dst, pad_g]).reshape(NW, CH, K)
    dst_s = jnp.concatenate([dst, pad_s]).reshape(NW, CH, K)

    z128 = jnp.zeros((RPS, D), jnp.float32)
    bias2d = bias.reshape(1, D)

    acc1, hist_dst = _sc_phase(x, src_g, dst_s, z128)
    out_e = _combine(acc1, hist_dst.transpose(1, 0, 2), bias2d, final=False)
    acc2, hist_src = _sc_phase(out_e, dst_g, src_s, z128)
    out = _combine(acc2, hist_src.transpose(1, 0, 2), bias2d, final=True)
    return out[:N]


# speedup vs baseline: 1.0900x; 1.0900x over previous
import functools

import jax
import jax.numpy as jnp
from jax import lax
from jax.experimental import pallas as pl
from jax.experimental.pallas import tpu as pltpu
from jax.experimental.pallas import tpu_sc as plsc

N = 10000
D = 128
NNZ = 320000
NC = 2
NS = 16
NW = NC * NS
K = 128
G = 16
CH = -(-NNZ // (NW * K * G)) * G
PER_W = CH * K
PAD = PER_W * NW
RPS = 632
NP = RPS * NS
HR = NP // 128
L = 16


def _sc_phase(table, gidx, sidx, z128):
    mesh = plsc.VectorSubcoreMesh(core_axis_name="c", subcore_axis_name="s")

    @functools.partial(
        pl.kernel,
        out_type=[
            jax.ShapeDtypeStruct((NC, NP, D), jnp.float32),
            jax.ShapeDtypeStruct((NW, HR, 128), jnp.float32),
        ],
        mesh=mesh,
        scratch_types=[
            pltpu.VMEM((G, K), jnp.int32),
            pltpu.VMEM((G, K), jnp.int32),
            pltpu.VMEM((2, K, D), jnp.float32),
            pltpu.VMEM((HR, 128), jnp.float32),
            pltpu.VMEM_SHARED((NP + 8, D), jnp.float32),
            pltpu.SemaphoreType.DMA,
            pltpu.SemaphoreType.DMA,
            pltpu.SemaphoreType.DMA,
        ],
        compiler_params=pltpu.CompilerParams(needs_layout_passes=False),
    )
    def phase(tab_hbm, gidx_hbm, sidx_hbm, z128_hbm,
              acc_out, hist_out, gi_v, si_v, rows_v, hist_v, acc_sp,
              gsem, ssem0, ssem1):
        c = lax.axis_index("c")
        s = lax.axis_index("s")
        w = s * NC + c
        base = s * RPS
        pltpu.sync_copy(z128_hbm, acc_sp.at[pl.ds(base, RPS)])

        zv = jnp.zeros((L,), jnp.float32)

        def zrow(r, carry):
            for l in range(128 // L):
                hist_v[r, pl.ds(l * L, L)] = zv
            return carry

        lax.fori_loop(0, HR, zrow, 0)
        plsc.subcore_barrier()

        ssem = (ssem0, ssem1)

        def group(g, carry):
            pltpu.sync_copy(gidx_hbm.at[w, pl.ds(g * G, G)], gi_v)
            pltpu.sync_copy(sidx_hbm.at[w, pl.ds(g * G, G)], si_v)
            gd = [None] * G
            sd = [None] * G
            gd[0] = pltpu.async_copy(tab_hbm.at[gi_v.at[0]], rows_v.at[0],
                                     gsem)
            for j in range(G):
                q = j & 1
                gd[j].wait()
                sd[j] = pltpu.async_copy(rows_v.at[q],
                                         acc_sp.at[si_v.at[j]],
                                         ssem[q], add=True)
                if j + 1 < G:
                    if j >= 1:
                        sd[j - 1].wait()
                    gd[j + 1] = pltpu.async_copy(tab_hbm.at[gi_v.at[j + 1]],
                                                 rows_v.at[1 - q], gsem)
                for u in range(K // L):
                    v = si_v[j, pl.ds(u * L, L)]
                    cnt, last = plsc.scan_count(v)
                    r = lax.shift_right_logical(v, 7)
                    col = lax.bitwise_and(v, 127)
                    plsc.addupdate_scatter(
                        hist_v, [r, col], cnt.astype(jnp.float32), mask=last)
            sd[G - 2].wait()
            sd[G - 1].wait()
            return carry

        lax.fori_loop(0, CH // G, group, 0)
        plsc.subcore_barrier()
        pltpu.sync_copy(acc_sp.at[pl.ds(base, RPS)],
                        acc_out.at[c, pl.ds(base, RPS)])
        pltpu.sync_copy(hist_v, hist_out.at[w])

    return phase(table, gidx, sidx, z128)


def _combine(acc_p, hist_p, bias2d, final):
    def body(a_ref, h_ref, b_ref, o_ref):
        sums = a_ref[0] + a_ref[1]
        cnt = jnp.sum(h_ref[0], axis=0)
        rec = jnp.where(cnt == 0.0, 0.0,
                        1.0 / jnp.where(cnt == 0.0, 1.0, cnt))
        ri = lax.broadcasted_iota(jnp.int32, (128, 128), 0)
        ci = lax.broadcasted_iota(jnp.int32, (128, 128), 1)
        dg = jnp.where(ri == ci, rec[None, :], 0.0)
        y = jax.lax.dot(dg, sums,
                        precision=jax.lax.Precision.HIGHEST,
                        preferred_element_type=jnp.float32)
        if final:
            y = y + b_ref[...]
            y = jnp.where(y >= 0.0, y, 0.01 * y)
        o_ref[...] = y

    return pl.pallas_call(
        body,
        grid=(HR,),
        in_specs=[
            pl.BlockSpec((NC, 128, D), lambda i: (0, i, 0)),
            pl.BlockSpec((1, NW, 128), lambda i: (i, 0, 0)),
            pl.BlockSpec((1, D), lambda i: (0, 0)),
        ],
        out_specs=pl.BlockSpec((128, D), lambda i: (i, 0)),
        out_shape=jax.ShapeDtypeStruct((NP, D), jnp.float32),
    )(acc_p, hist_p, bias2d)


def kernel(x, hyperedge_index, bias):
    src = hyperedge_index[0].astype(jnp.int32)
    dst = hyperedge_index[1].astype(jnp.int32)
    npad = PAD - NNZ
    pad_s = N + (jnp.arange(npad, dtype=jnp.int32) % (NP - N))
    pad_g = jnp.zeros((npad,), jnp.int32)
    src_g = jnp.concatenate([src, pad_g]).reshape(NW, CH, K)
    src_s = jnp.concatenate([src, pad_s]).reshape(NW, CH, K)
    dst_g = jnp.concatenate([dst, pad_g]).reshape(NW, CH, K)
    dst_s = jnp.concatenate([dst, pad_s]).reshape(NW, CH, K)

    z128 = jnp.zeros((RPS, D), jnp.float32)
    bias2d = bias.reshape(1, D)

    acc1, hist_dst = _sc_phase(x, src_g, dst_s, z128)
    out_e = _combine(acc1, hist_dst.transpose(1, 0, 2), bias2d, final=False)
    acc2, hist_src = _sc_phase(out_e, dst_g, src_s, z128)
    out = _combine(acc2, hist_src.transpose(1, 0, 2), bias2d, final=True)
    return out[:N]


# speedup vs baseline: 1.1853x; 1.0874x over previous
import functools

import jax
import jax.numpy as jnp
from jax import lax
from jax.experimental import pallas as pl
from jax.experimental.pallas import tpu as pltpu
from jax.experimental.pallas import tpu_sc as plsc

N = 10000
D = 128
NNZ = 320000
NC = 2
NS = 16
NW = NC * NS
K = 128
G = 16
CH = -(-NNZ // (NW * K * G)) * G
PAD = CH * K * NW
TCH = PAD // K
CH0 = 112
CH1 = TCH // NS - CH0
RPS = 632
NP = RPS * NS
HR = NP // 128
L = 16


def _sc_phase(table, gidx, sidx, z128):
    mesh = plsc.VectorSubcoreMesh(core_axis_name="c", subcore_axis_name="s")

    @functools.partial(
        pl.kernel,
        out_type=[
            jax.ShapeDtypeStruct((NC, NP, D), jnp.float32),
            jax.ShapeDtypeStruct((NW, HR, 128), jnp.float32),
        ],
        mesh=mesh,
        scratch_types=[
            pltpu.VMEM((G, K), jnp.int32),
            pltpu.VMEM((G, K), jnp.int32),
            pltpu.VMEM((2, K, D), jnp.float32),
            pltpu.VMEM((HR, 128), jnp.float32),
            pltpu.VMEM_SHARED((NP + 8, D), jnp.float32),
            pltpu.SemaphoreType.DMA,
            pltpu.SemaphoreType.DMA,
            pltpu.SemaphoreType.DMA,
        ],
        compiler_params=pltpu.CompilerParams(needs_layout_passes=False),
    )
    def phase(tab_hbm, gidx_hbm, sidx_hbm, z128_hbm,
              acc_out, hist_out, gi_v, si_v, rows_v, hist_v, acc_sp,
              gsem, ssem0, ssem1):
        c = lax.axis_index("c")
        s = lax.axis_index("s")
        base = s * RPS
        cb = jnp.where(c == 0, s * CH0, NS * CH0 + s * CH1)
        ng = jnp.where(c == 0, CH0 // G, CH1 // G)
        w = s * NC + c
        pltpu.sync_copy(z128_hbm, acc_sp.at[pl.ds(base, RPS)])

        zv = jnp.zeros((L,), jnp.float32)

        def zrow(r, carry):
            for l in range(128 // L):
                hist_v[r, pl.ds(l * L, L)] = zv
            return carry

        lax.fori_loop(0, HR, zrow, 0)
        plsc.subcore_barrier()

        ssem = (ssem0, ssem1)

        def group(g, carry):
            pltpu.sync_copy(gidx_hbm.at[pl.ds(cb + g * G, G)], gi_v)
            pltpu.sync_copy(sidx_hbm.at[pl.ds(cb + g * G, G)], si_v)
            gd = [None] * G
            sd = [None] * G
            gd[0] = pltpu.async_copy(tab_hbm.at[gi_v.at[0]], rows_v.at[0],
                                     gsem)
            for j in range(G):
                q = j & 1
                gd[j].wait()
                sd[j] = pltpu.async_copy(rows_v.at[q],
                                         acc_sp.at[si_v.at[j]],
                                         ssem[q], add=True)
                if j + 1 < G:
                    if j >= 1:
                        sd[j - 1].wait()
                    gd[j + 1] = pltpu.async_copy(tab_hbm.at[gi_v.at[j + 1]],
                                                 rows_v.at[1 - q], gsem)
                for u in range(K // L):
                    v = si_v[j, pl.ds(u * L, L)]
                    cnt, last = plsc.scan_count(v)
                    r = lax.shift_right_logical(v, 7)
                    col = lax.bitwise_and(v, 127)
                    plsc.addupdate_scatter(
                        hist_v, [r, col], cnt.astype(jnp.float32), mask=last)
            sd[G - 2].wait()
            sd[G - 1].wait()
            return carry

        lax.fori_loop(0, ng, group, 0)
        plsc.subcore_barrier()
        pltpu.sync_copy(acc_sp.at[pl.ds(base, RPS)],
                        acc_out.at[c, pl.ds(base, RPS)])
        pltpu.sync_copy(hist_v, hist_out.at[w])

    return phase(table, gidx, sidx, z128)


def _combine(acc_p, hist_p, bias2d, final):
    def body(a_ref, h_ref, b_ref, o_ref):
        sums = a_ref[0] + a_ref[1]
        cnt = jnp.sum(h_ref[0], axis=0)
        rec = jnp.where(cnt == 0.0, 0.0,
                        1.0 / jnp.where(cnt == 0.0, 1.0, cnt))
        ri = lax.broadcasted_iota(jnp.int32, (128, 128), 0)
        ci = lax.broadcasted_iota(jnp.int32, (128, 128), 1)
        dg = jnp.where(ri == ci, rec[None, :], 0.0)
        y = jax.lax.dot(dg, sums,
                        precision=jax.lax.Precision.HIGHEST,
                        preferred_element_type=jnp.float32)
        if final:
            y = y + b_ref[...]
            y = jnp.where(y >= 0.0, y, 0.01 * y)
        o_ref[...] = y

    return pl.pallas_call(
        body,
        grid=(HR,),
        in_specs=[
            pl.BlockSpec((NC, 128, D), lambda i: (0, i, 0)),
            pl.BlockSpec((1, NW, 128), lambda i: (i, 0, 0)),
            pl.BlockSpec((1, D), lambda i: (0, 0)),
        ],
        out_specs=pl.BlockSpec((128, D), lambda i: (i, 0)),
        out_shape=jax.ShapeDtypeStruct((NP, D), jnp.float32),
    )(acc_p, hist_p, bias2d)


def kernel(x, hyperedge_index, bias):
    src = hyperedge_index[0].astype(jnp.int32)
    dst = hyperedge_index[1].astype(jnp.int32)
    npad = PAD - NNZ
    pad_s = N + (jnp.arange(npad, dtype=jnp.int32) % (NP - N))
    pad_g = jnp.zeros((npad,), jnp.int32)
    src_g = jnp.concatenate([src, pad_g]).reshape(TCH, K)
    src_s = jnp.concatenate([src, pad_s]).reshape(TCH, K)
    dst_g = jnp.concatenate([dst, pad_g]).reshape(TCH, K)
    dst_s = jnp.concatenate([dst, pad_s]).reshape(TCH, K)

    z128 = jnp.zeros((RPS, D), jnp.float32)
    bias2d = bias.reshape(1, D)

    acc1, hist_dst = _sc_phase(x, src_g, dst_s, z128)
    out_e = _combine(acc1, hist_dst.transpose(1, 0, 2), bias2d, final=False)
    acc2, hist_src = _sc_phase(out_e, dst_g, src_s, z128)
    out = _combine(acc2, hist_src.transpose(1, 0, 2), bias2d, final=True)
    return out[:N]


# speedup vs baseline: 1.2012x; 1.0134x over previous
import functools

import jax
import jax.numpy as jnp
from jax import lax
from jax.experimental import pallas as pl
from jax.experimental.pallas import tpu as pltpu
from jax.experimental.pallas import tpu_sc as plsc

N = 10000
D = 128
NNZ = 320000
NC = 2
NS = 16
NW = NC * NS
K = 128
G = 16
CH = -(-NNZ // (NW * K * G)) * G
PAD = CH * K * NW
TCH = PAD // K
CH0 = 112
CH1 = TCH // NS - CH0
RPS = 632
NP = RPS * NS
HR = NP // 128
L = 16


def _sc_phase(table, gidx, sidx):
    mesh = plsc.VectorSubcoreMesh(core_axis_name="c", subcore_axis_name="s")

    @functools.partial(
        pl.kernel,
        out_type=[
            jax.ShapeDtypeStruct((NC, NP, D), jnp.float32),
            jax.ShapeDtypeStruct((NW, HR, 128), jnp.float32),
        ],
        mesh=mesh,
        scratch_types=[
            pltpu.VMEM((G, K), jnp.int32),
            pltpu.VMEM((G, K), jnp.int32),
            pltpu.VMEM((2, K, D), jnp.float32),
            pltpu.VMEM((HR, 128), jnp.float32),
            pltpu.VMEM_SHARED((NP + 8, D), jnp.float32),
            pltpu.SemaphoreType.DMA,
            pltpu.SemaphoreType.DMA,
            pltpu.SemaphoreType.DMA,
        ],
        compiler_params=pltpu.CompilerParams(needs_layout_passes=False),
    )
    def phase(tab_hbm, gidx_hbm, sidx_hbm,
              acc_out, hist_out, gi_v, si_v, rows_v, hist_v, acc_sp,
              gsem, ssem0, ssem1):
        c = lax.axis_index("c")
        s = lax.axis_index("s")
        base = s * RPS
        cb = jnp.where(c == 0, s * CH0, NS * CH0 + s * CH1)
        ng = jnp.where(c == 0, CH0 // G, CH1 // G)
        w = s * NC + c
        zv = jnp.zeros((L,), jnp.float32)

        def zrow(r, carry):
            for l in range(128 // L):
                hist_v[r, pl.ds(l * L, L)] = zv
            return carry

        lax.fori_loop(0, HR, zrow, 0)
        for t in range(RPS // HR):
            pltpu.sync_copy(hist_v, acc_sp.at[pl.ds(base + t * HR, HR)])
        plsc.subcore_barrier()

        ssem = (ssem0, ssem1)

        def group(g, carry):
            pltpu.sync_copy(gidx_hbm.at[pl.ds(cb + g * G, G)], gi_v)
            pltpu.sync_copy(sidx_hbm.at[pl.ds(cb + g * G, G)], si_v)
            gd = [None] * G
            sd = [None] * G
            gd[0] = pltpu.async_copy(tab_hbm.at[gi_v.at[0]], rows_v.at[0],
                                     gsem)
            for j in range(G):
                q = j & 1
                gd[j].wait()
                sd[j] = pltpu.async_copy(rows_v.at[q],
                                         acc_sp.at[si_v.at[j]],
                                         ssem[q], add=True)
                if j + 1 < G:
                    if j >= 1:
                        sd[j - 1].wait()
                    gd[j + 1] = pltpu.async_copy(tab_hbm.at[gi_v.at[j + 1]],
                                                 rows_v.at[1 - q], gsem)
                for u in range(K // L):
                    v = si_v[j, pl.ds(u * L, L)]
                    cnt, last = plsc.scan_count(v)
                    r = lax.shift_right_logical(v, 7)
                    col = lax.bitwise_and(v, 127)
                    plsc.addupdate_scatter(
                        hist_v, [r, col], cnt.astype(jnp.float32), mask=last)
            sd[G - 2].wait()
            sd[G - 1].wait()
            return carry

        lax.fori_loop(0, ng, group, 0)
        plsc.subcore_barrier()
        pltpu.sync_copy(acc_sp.at[pl.ds(base, RPS)],
                        acc_out.at[c, pl.ds(base, RPS)])
        pltpu.sync_copy(hist_v, hist_out.at[w])

    return phase(table, gidx, sidx)


def _combine(acc_p, hist_p, bias2d, final):
    def body(a_ref, h_ref, b_ref, o_ref):
        sums = a_ref[0] + a_ref[1]
        cnt = jnp.sum(h_ref[0], axis=0)
        rec = jnp.where(cnt == 0.0, 0.0,
                        1.0 / jnp.where(cnt == 0.0, 1.0, cnt))
        ri = lax.broadcasted_iota(jnp.int32, (128, 128), 0)
        ci = lax.broadcasted_iota(jnp.int32, (128, 128), 1)
        dg = jnp.where(ri == ci, rec[None, :], 0.0)
        y = jax.lax.dot(dg, sums,
                        precision=jax.lax.Precision.HIGHEST,
                        preferred_element_type=jnp.float32)
        if final:
            y = y + b_ref[...]
            y = jnp.where(y >= 0.0, y, 0.01 * y)
        o_ref[...] = y

    return pl.pallas_call(
        body,
        grid=(HR,),
        in_specs=[
            pl.BlockSpec((NC, 128, D), lambda i: (0, i, 0)),
            pl.BlockSpec((1, NW, 128), lambda i: (i, 0, 0)),
            pl.BlockSpec((1, D), lambda i: (0, 0)),
        ],
        out_specs=pl.BlockSpec((128, D), lambda i: (i, 0)),
        out_shape=jax.ShapeDtypeStruct((NP, D), jnp.float32),
    )(acc_p, hist_p, bias2d)


def kernel(x, hyperedge_index, bias):
    src = hyperedge_index[0].astype(jnp.int32)
    dst = hyperedge_index[1].astype(jnp.int32)
    npad = PAD - NNZ
    pad_s = N + (jnp.arange(npad, dtype=jnp.int32) % (NP - N))
    pad_g = jnp.zeros((npad,), jnp.int32)
    src_g = jnp.concatenate([src, pad_g]).reshape(TCH, K)
    src_s = jnp.concatenate([src, pad_s]).reshape(TCH, K)
    dst_g = jnp.concatenate([dst, pad_g]).reshape(TCH, K)
    dst_s = jnp.concatenate([dst, pad_s]).reshape(TCH, K)

    bias2d = bias.reshape(1, D)

    acc1, hist_dst = _sc_phase(x, src_g, dst_s)
    out_e = _combine(acc1, hist_dst.transpose(1, 0, 2), bias2d, final=False)
    acc2, hist_src = _sc_phase(out_e, dst_g, src_s)
    out = _combine(acc2, hist_src.transpose(1, 0, 2), bias2d, final=True)
    return out[:N]


# speedup vs baseline: 1.2475x; 1.0385x over previous
import functools

import jax
import jax.numpy as jnp
from jax import lax
from jax.experimental import pallas as pl
from jax.experimental.pallas import tpu as pltpu
from jax.experimental.pallas import tpu_sc as plsc

N = 10000
D = 128
NNZ = 320000
NC = 2
NS = 16
NW = NC * NS
K = 128
G = 16
CH = -(-NNZ // (NW * K * G)) * G
PAD = CH * K * NW
TCH = PAD // K
CH0 = 128
CH1 = TCH // NS - CH0
RPS = 632
NP = RPS * NS
HR = NP // 128
L = 16


def _sc_phase(table, gidx, sidx):
    mesh = plsc.VectorSubcoreMesh(core_axis_name="c", subcore_axis_name="s")

    @functools.partial(
        pl.kernel,
        out_type=[
            jax.ShapeDtypeStruct((NC, NP, D), jnp.float32),
            jax.ShapeDtypeStruct((NW, HR, 128), jnp.float32),
        ],
        mesh=mesh,
        scratch_types=[
            pltpu.VMEM((G, K), jnp.int32),
            pltpu.VMEM((G, K), jnp.int32),
            pltpu.VMEM((2, K, D), jnp.float32),
            pltpu.VMEM((HR, 128), jnp.float32),
            pltpu.VMEM_SHARED((NP + 8, D), jnp.float32),
            pltpu.SemaphoreType.DMA,
            pltpu.SemaphoreType.DMA,
            pltpu.SemaphoreType.DMA,
        ],
        compiler_params=pltpu.CompilerParams(needs_layout_passes=False),
    )
    def phase(tab_hbm, gidx_hbm, sidx_hbm,
              acc_out, hist_out, gi_v, si_v, rows_v, hist_v, acc_sp,
              gsem, ssem0, ssem1):
        c = lax.axis_index("c")
        s = lax.axis_index("s")
        base = s * RPS
        cb = jnp.where(c == 0, s * CH0, NS * CH0 + s * CH1)
        ng = jnp.where(c == 0, CH0 // G, CH1 // G)
        w = s * NC + c
        zv = jnp.zeros((L,), jnp.float32)

        def zrow(r, carry):
            for l in range(128 // L):
                hist_v[r, pl.ds(l * L, L)] = zv
            return carry

        lax.fori_loop(0, HR, zrow, 0)
        for t in range(RPS // HR):
            pltpu.sync_copy(hist_v, acc_sp.at[pl.ds(base + t * HR, HR)])
        plsc.subcore_barrier()

        ssem = (ssem0, ssem1)

        def group(g, carry):
            pltpu.sync_copy(gidx_hbm.at[pl.ds(cb + g * G, G)], gi_v)
            pltpu.sync_copy(sidx_hbm.at[pl.ds(cb + g * G, G)], si_v)
            gd = [None] * G
            sd = [None] * G
            gd[0] = pltpu.async_copy(tab_hbm.at[gi_v.at[0]], rows_v.at[0],
                                     gsem)
            for j in range(G):
                q = j & 1
                gd[j].wait()
                sd[j] = pltpu.async_copy(rows_v.at[q],
                                         acc_sp.at[si_v.at[j]],
                                         ssem[q], add=True)
                if j + 1 < G:
                    if j >= 1:
                        sd[j - 1].wait()
                    gd[j + 1] = pltpu.async_copy(tab_hbm.at[gi_v.at[j + 1]],
                                                 rows_v.at[1 - q], gsem)
                for u in range(K // L):
                    v = si_v[j, pl.ds(u * L, L)]
                    cnt, last = plsc.scan_count(v)
                    r = lax.shift_right_logical(v, 7)
                    col = lax.bitwise_and(v, 127)
                    plsc.addupdate_scatter(
                        hist_v, [r, col], cnt.astype(jnp.float32), mask=last)
            sd[G - 2].wait()
            sd[G - 1].wait()
            return carry

        lax.fori_loop(0, ng, group, 0)
        plsc.subcore_barrier()
        pltpu.sync_copy(acc_sp.at[pl.ds(base, RPS)],
                        acc_out.at[c, pl.ds(base, RPS)])
        pltpu.sync_copy(hist_v, hist_out.at[w])

    return phase(table, gidx, sidx)


def _combine(acc_p, hist_p, bias2d, final):
    def body(a_ref, h_ref, b_ref, o_ref):
        sums = a_ref[0] + a_ref[1]
        cnt = jnp.sum(h_ref[0], axis=0)
        rec = jnp.where(cnt == 0.0, 0.0,
                        1.0 / jnp.where(cnt == 0.0, 1.0, cnt))
        ri = lax.broadcasted_iota(jnp.int32, (128, 128), 0)
        ci = lax.broadcasted_iota(jnp.int32, (128, 128), 1)
        dg = jnp.where(ri == ci, rec[None, :], 0.0)
        y = jax.lax.dot(dg, sums,
                        precision=jax.lax.Precision.HIGHEST,
                        preferred_element_type=jnp.float32)
        if final:
            y = y + b_ref[...]
            y = jnp.where(y >= 0.0, y, 0.01 * y)
        o_ref[...] = y

    return pl.pallas_call(
        body,
        grid=(HR,),
        in_specs=[
            pl.BlockSpec((NC, 128, D), lambda i: (0, i, 0)),
            pl.BlockSpec((1, NW, 128), lambda i: (i, 0, 0)),
            pl.BlockSpec((1, D), lambda i: (0, 0)),
        ],
        out_specs=pl.BlockSpec((128, D), lambda i: (i, 0)),
        out_shape=jax.ShapeDtypeStruct((NP, D), jnp.float32),
    )(acc_p, hist_p, bias2d)


def kernel(x, hyperedge_index, bias):
    src = hyperedge_index[0].astype(jnp.int32)
    dst = hyperedge_index[1].astype(jnp.int32)
    npad = PAD - NNZ
    pad_s = N + (jnp.arange(npad, dtype=jnp.int32) % (NP - N))
    pad_g = jnp.zeros((npad,), jnp.int32)
    src_g = jnp.concatenate([src, pad_g]).reshape(TCH, K)
    src_s = jnp.concatenate([src, pad_s]).reshape(TCH, K)
    dst_g = jnp.concatenate([dst, pad_g]).reshape(TCH, K)
    dst_s = jnp.concatenate([dst, pad_s]).reshape(TCH, K)

    bias2d = bias.reshape(1, D)

    acc1, hist_dst = _sc_phase(x, src_g, dst_s)
    out_e = _combine(acc1, hist_dst.transpose(1, 0, 2), bias2d, final=False)
    acc2, hist_src = _sc_phase(out_e, dst_g, src_s)
    out = _combine(acc2, hist_src.transpose(1, 0, 2), bias2d, final=True)
    return out[:N]


# speedup vs baseline: 1.4187x; 1.1372x over previous
import functools

import jax
import jax.numpy as jnp
from jax import lax
from jax.experimental import pallas as pl
from jax.experimental.pallas import tpu as pltpu
from jax.experimental.pallas import tpu_sc as plsc

N = 10000
D = 128
NNZ = 320000
NC = 2
NS = 16
NW = NC * NS
K = 128
G = 16
CH = -(-NNZ // (NW * K * G)) * G
PAD = CH * K * NW
TCH = PAD // K
CH0 = 144
CH1 = TCH // NS - CH0
RPS = 632
NP = RPS * NS
HR = NP // 128
L = 16


def _sc_phase(table, gidx, sidx):
    mesh = plsc.VectorSubcoreMesh(core_axis_name="c", subcore_axis_name="s")

    @functools.partial(
        pl.kernel,
        out_type=[
            jax.ShapeDtypeStruct((NC, NP, D), jnp.float32),
            jax.ShapeDtypeStruct((NW, HR, 128), jnp.float32),
        ],
        mesh=mesh,
        scratch_types=[
            pltpu.VMEM((G, K), jnp.int32),
            pltpu.VMEM((G, K), jnp.int32),
            pltpu.VMEM((2, K, D), jnp.float32),
            pltpu.VMEM((HR, 128), jnp.float32),
            pltpu.VMEM_SHARED((NP + 8, D), jnp.float32),
            pltpu.SemaphoreType.DMA,
            pltpu.SemaphoreType.DMA,
            pltpu.SemaphoreType.DMA,
        ],
        compiler_params=pltpu.CompilerParams(needs_layout_passes=False),
    )
    def phase(tab_hbm, gidx_hbm, sidx_hbm,
              acc_out, hist_out, gi_v, si_v, rows_v, hist_v, acc_sp,
              gsem, ssem0, ssem1):
        c = lax.axis_index("c")
        s = lax.axis_index("s")
        base = s * RPS
        cb = jnp.where(c == 0, s * CH0, NS * CH0 + s * CH1)
        ng = jnp.where(c == 0, CH0 // G, CH1 // G)
        w = s * NC + c
        zv = jnp.zeros((L,), jnp.float32)

        def zrow(r, carry):
            for l in range(128 // L):
                hist_v[r, pl.ds(l * L, L)] = zv
            return carry

        lax.fori_loop(0, HR, zrow, 0)
        for t in range(RPS // HR):
            pltpu.sync_copy(hist_v, acc_sp.at[pl.ds(base + t * HR, HR)])
        plsc.subcore_barrier()

        ssem = (ssem0, ssem1)

        def group(g, carry):
            pltpu.sync_copy(gidx_hbm.at[pl.ds(cb + g * G, G)], gi_v)
            pltpu.sync_copy(sidx_hbm.at[pl.ds(cb + g * G, G)], si_v)
            gd = [None] * G
            sd = [None] * G
            gd[0] = pltpu.async_copy(tab_hbm.at[gi_v.at[0]], rows_v.at[0],
                                     gsem)
            for j in range(G):
                q = j & 1
                gd[j].wait()
                sd[j] = pltpu.async_copy(rows_v.at[q],
                                         acc_sp.at[si_v.at[j]],
                                         ssem[q], add=True)
                if j + 1 < G:
                    if j >= 1:
                        sd[j - 1].wait()
                    gd[j + 1] = pltpu.async_copy(tab_hbm.at[gi_v.at[j + 1]],
                                                 rows_v.at[1 - q], gsem)
                for u in range(K // L):
                    v = si_v[j, pl.ds(u * L, L)]
                    cnt, last = plsc.scan_count(v)
                    r = lax.shift_right_logical(v, 7)
                    col = lax.bitwise_and(v, 127)
                    plsc.addupdate_scatter(
                        hist_v, [r, col], cnt.astype(jnp.float32), mask=last)
            sd[G - 2].wait()
            sd[G - 1].wait()
            return carry

        lax.fori_loop(0, ng, group, 0)
        plsc.subcore_barrier()
        pltpu.sync_copy(acc_sp.at[pl.ds(base, RPS)],
                        acc_out.at[c, pl.ds(base, RPS)])
        pltpu.sync_copy(hist_v, hist_out.at[w])

    return phase(table, gidx, sidx)


def _combine(acc_p, hist_p, bias2d, final):
    def body(a_ref, h_ref, b_ref, o_ref):
        sums = a_ref[0] + a_ref[1]
        cnt = jnp.sum(h_ref[0], axis=0)
        rec = jnp.where(cnt == 0.0, 0.0,
                        1.0 / jnp.where(cnt == 0.0, 1.0, cnt))
        ri = lax.broadcasted_iota(jnp.int32, (128, 128), 0)
        ci = lax.broadcasted_iota(jnp.int32, (128, 128), 1)
        dg = jnp.where(ri == ci, rec[None, :], 0.0)
        y = jax.lax.dot(dg, sums,
                        precision=jax.lax.Precision.HIGHEST,
                        preferred_element_type=jnp.float32)
        if final:
            y = y + b_ref[...]
            y = jnp.where(y >= 0.0, y, 0.01 * y)
        o_ref[...] = y

    return pl.pallas_call(
        body,
        grid=(HR,),
        in_specs=[
            pl.BlockSpec((NC, 128, D), lambda i: (0, i, 0)),
            pl.BlockSpec((1, NW, 128), lambda i: (i, 0, 0)),
            pl.BlockSpec((1, D), lambda i: (0, 0)),
        ],
        out_specs=pl.BlockSpec((128, D), lambda i: (i, 0)),
        out_shape=jax.ShapeDtypeStruct((NP, D), jnp.float32),
    )(acc_p, hist_p, bias2d)


def kernel(x, hyperedge_index, bias):
    src = hyperedge_index[0].astype(jnp.int32)
    dst = hyperedge_index[1].astype(jnp.int32)
    npad = PAD - NNZ
    pad_s = N + (jnp.arange(npad, dtype=jnp.int32) % (NP - N))
    pad_g = jnp.zeros((npad,), jnp.int32)
    src_g = jnp.concatenate([src, pad_g]).reshape(TCH, K)
    src_s = jnp.concatenate([src, pad_s]).reshape(TCH, K)
    dst_g = jnp.concatenate([dst, pad_g]).reshape(TCH, K)
    dst_s = jnp.concatenate([dst, pad_s]).reshape(TCH, K)

    bias2d = bias.reshape(1, D)

    acc1, hist_dst = _sc_phase(x, src_g, dst_s)
    out_e = _combine(acc1, hist_dst.transpose(1, 0, 2), bias2d, final=False)
    acc2, hist_src = _sc_phase(out_e, dst_g, src_s)
    out = _combine(acc2, hist_src.transpose(1, 0, 2), bias2d, final=True)
    return out[:N]
